# Initial kernel scaffold; baseline (speedup 1.0000x reference)
import jax
import jax.numpy as jnp
from jax.experimental import pallas as pl


def kernel(tokens, W_E):
    raise NotImplementedError("write your pallas kernel here")


# speedup vs baseline: 1.4647x; 1.4647x over previous
import functools

import jax
import jax.numpy as jnp
from jax import lax
from jax.experimental import pallas as pl
from jax.experimental.pallas import tpu as pltpu
from jax.experimental.pallas import tpu_sc as plsc

_NC = 2
_NS = 16
_NW = _NC * _NS

_B = 4 * 2048
_D = 768
_B_PER_W = _B // _NW
_CH = 64
_NCH = _B_PER_W // _CH


def _embed_lookup(tokens_w, w_e):
    mesh = plsc.VectorSubcoreMesh(core_axis_name="c", subcore_axis_name="s")

    @functools.partial(
        pl.kernel,
        mesh=mesh,
        out_type=jax.ShapeDtypeStruct((_B, _D), jnp.float32),
        scratch_types=[
            pltpu.VMEM((_NCH, _CH), jnp.int32),
            pltpu.VMEM((_CH, _D), jnp.float32),
            pltpu.VMEM((_CH, _D), jnp.float32),
            pltpu.SemaphoreType.DMA,
            pltpu.SemaphoreType.DMA,
            pltpu.SemaphoreType.DMA,
            pltpu.SemaphoreType.DMA,
        ],
    )
    def body(tok_hbm, table_hbm, out_hbm, idx_v, buf0, buf1,
             gsem0, gsem1, osem0, osem1):
        wid = lax.axis_index("s") * _NC + lax.axis_index("c")
        base = wid * _B_PER_W
        pltpu.sync_copy(tok_hbm.at[wid], idx_v)

        bufs = (buf0, buf1)
        gsems = (gsem0, gsem1)
        osems = (osem0, osem1)
        gh = [None, None]
        oh = [None, None]
        gh[0] = pltpu.async_copy(table_hbm.at[idx_v.at[0]], bufs[0], gsems[0])
        gh[1] = pltpu.async_copy(table_hbm.at[idx_v.at[1]], bufs[1], gsems[1])
        for c in range(_NCH):
            b = c & 1
            gh[b].wait()
            oh[b] = pltpu.async_copy(
                bufs[b], out_hbm.at[pl.ds(base + c * _CH, _CH)], osems[b])
            nxt = c + 2
            if nxt < _NCH:
                oh[b].wait()
                gh[b] = pltpu.async_copy(
                    table_hbm.at[idx_v.at[nxt]], bufs[b], gsems[b])
        oh[0].wait()
        oh[1].wait()

    return body(tokens_w, w_e)


def kernel(tokens, W_E):
    batch, seq = tokens.shape
    tokens_w = tokens.reshape(_NW, _NCH, _CH)
    out = _embed_lookup(tokens_w, W_E)
    return out.reshape(batch, seq, _D)


# speedup vs baseline: 1.5087x; 1.0300x over previous
import functools

import jax
import jax.numpy as jnp
from jax import lax
from jax.experimental import pallas as pl
from jax.experimental.pallas import tpu as pltpu
from jax.experimental.pallas import tpu_sc as plsc

_NC = 2
_NS = 16
_NW = _NC * _NS

_B = 4 * 2048
_D = 768
_B_PER_W = _B // _NW
_CH = 32
_NCH = _B_PER_W // _CH
_NBUF = 4


def _embed_lookup(tokens_w, w_e):
    mesh = plsc.VectorSubcoreMesh(core_axis_name="c", subcore_axis_name="s")

    @functools.partial(
        pl.kernel,
        mesh=mesh,
        out_type=jax.ShapeDtypeStruct((_B, _D), jnp.float32),
        scratch_types=(
            [pltpu.VMEM((_NCH, _CH), jnp.int32)]
            + [pltpu.VMEM((_CH, _D), jnp.float32) for _ in range(_NBUF)]
            + [pltpu.SemaphoreType.DMA for _ in range(2 * _NBUF)]
        ),
    )
    def body(tok_hbm, table_hbm, out_hbm, idx_v, *rest):
        bufs = rest[:_NBUF]
        gsems = rest[_NBUF:2 * _NBUF]
        osems = rest[2 * _NBUF:]
        wid = lax.axis_index("s") * _NC + lax.axis_index("c")
        base = wid * _B_PER_W
        pltpu.sync_copy(tok_hbm.at[wid], idx_v)

        gh = [None] * _NBUF
        oh = [None] * _NBUF
        for b in range(_NBUF):
            gh[b] = pltpu.async_copy(table_hbm.at[idx_v.at[b]], bufs[b],
                                     gsems[b])
        for c in range(_NCH):
            b = c % _NBUF
            gh[b].wait()
            oh[b] = pltpu.async_copy(
                bufs[b], out_hbm.at[pl.ds(base + c * _CH, _CH)], osems[b])
            nxt = c + _NBUF
            if nxt < _NCH:
                oh[b].wait()
                gh[b] = pltpu.async_copy(
                    table_hbm.at[idx_v.at[nxt]], bufs[b], gsems[b])
        for b in range(_NBUF):
            oh[b].wait()

    return body(tokens_w, w_e)


def kernel(tokens, W_E):
    batch, seq = tokens.shape
    tokens_w = tokens.reshape(_NW, _NCH, _CH)
    out = _embed_lookup(tokens_w, W_E)
    return out.reshape(batch, seq, _D)


# speedup vs baseline: 1.5225x; 1.0092x over previous
import functools

import jax
import jax.numpy as jnp
from jax import lax
from jax.experimental import pallas as pl
from jax.experimental.pallas import tpu as pltpu
from jax.experimental.pallas import tpu_sc as plsc

_NC = 2
_NS = 16
_NW = _NC * _NS

_BATCH = 4
_SEQ = 2048
_D = 768
_B = _BATCH * _SEQ
_B_PER_W = _B // _NW
_W_PER_ROW = _SEQ // _B_PER_W
_CH = 32
_NCH = _B_PER_W // _CH
_NBUF = 4


def _embed_lookup(tokens, w_e):
    mesh = plsc.VectorSubcoreMesh(core_axis_name="c", subcore_axis_name="s")

    @functools.partial(
        pl.kernel,
        mesh=mesh,
        out_type=jax.ShapeDtypeStruct((_BATCH, _SEQ, _D), jnp.float32),
        scratch_types=(
            [pltpu.VMEM((_B_PER_W,), jnp.int32)]
            + [pltpu.VMEM((_CH, _D), jnp.float32) for _ in range(_NBUF)]
            + [pltpu.SemaphoreType.DMA for _ in range(2 * _NBUF)]
        ),
    )
    def body(tok_hbm, table_hbm, out_hbm, idx_v, *rest):
        bufs = rest[:_NBUF]
        gsems = rest[_NBUF:2 * _NBUF]
        osems = rest[2 * _NBUF:]
        wid = lax.axis_index("s") * _NC + lax.axis_index("c")
        row = wid // _W_PER_ROW
        col0 = (wid % _W_PER_ROW) * _B_PER_W
        pltpu.sync_copy(tok_hbm.at[row, pl.ds(col0, _B_PER_W)], idx_v)

        gh = [None] * _NBUF
        oh = [None] * _NBUF
        for b in range(_NBUF):
            gh[b] = pltpu.async_copy(
                table_hbm.at[idx_v.at[pl.ds(b * _CH, _CH)]], bufs[b], gsems[b])
        for c in range(_NCH):
            b = c % _NBUF
            gh[b].wait()
            oh[b] = pltpu.async_copy(
                bufs[b], out_hbm.at[row, pl.ds(col0 + c * _CH, _CH)], osems[b])
            nxt = c + _NBUF
            if nxt < _NCH:
                oh[b].wait()
                gh[b] = pltpu.async_copy(
                    table_hbm.at[idx_v.at[pl.ds(nxt * _CH, _CH)]], bufs[b],
                    gsems[b])
        for b in range(_NBUF):
            oh[b].wait()

    return body(tokens, w_e)


def kernel(tokens, W_E):
    return _embed_lookup(tokens, W_E)


# speedup vs baseline: 1.5314x; 1.0058x over previous
import functools

import jax
import jax.numpy as jnp
from jax import lax
from jax.experimental import pallas as pl
from jax.experimental.pallas import tpu as pltpu
from jax.experimental.pallas import tpu_sc as plsc

_NC = 2
_NS = 16
_NW = _NC * _NS

_BATCH = 4
_SEQ = 2048
_D = 768
_B = _BATCH * _SEQ
_B_PER_W = _B // _NW
_W_PER_ROW = _SEQ // _B_PER_W
_CH = 32
_NCH = _B_PER_W // _CH
_NBUF = 5


def _embed_lookup(tokens, w_e):
    mesh = plsc.VectorSubcoreMesh(core_axis_name="c", subcore_axis_name="s")

    @functools.partial(
        pl.kernel,
        mesh=mesh,
        out_type=jax.ShapeDtypeStruct((_BATCH, _SEQ, _D), jnp.float32),
        scratch_types=(
            [pltpu.VMEM((_B_PER_W,), jnp.int32)]
            + [pltpu.VMEM((_CH, _D), jnp.float32) for _ in range(_NBUF)]
            + [pltpu.SemaphoreType.DMA for _ in range(2 * _NBUF)]
        ),
    )
    def body(tok_hbm, table_hbm, out_hbm, idx_v, *rest):
        bufs = rest[:_NBUF]
        gsems = rest[_NBUF:2 * _NBUF]
        osems = rest[2 * _NBUF:]
        wid = lax.axis_index("s") * _NC + lax.axis_index("c")
        row = wid // _W_PER_ROW
        col0 = (wid % _W_PER_ROW) * _B_PER_W
        pltpu.sync_copy(tok_hbm.at[row, pl.ds(col0, _B_PER_W)], idx_v)

        gh = [None] * _NBUF
        oh = [None] * _NBUF
        for b in range(_NBUF):
            gh[b] = pltpu.async_copy(
                table_hbm.at[idx_v.at[pl.ds(b * _CH, _CH)]], bufs[b], gsems[b])
        for c in range(_NCH):
            b = c % _NBUF
            if c >= 1 and c - 1 + _NBUF < _NCH:
                pb = (c - 1) % _NBUF
                oh[pb].wait()
                gh[pb] = pltpu.async_copy(
                    table_hbm.at[idx_v.at[pl.ds((c - 1 + _NBUF) * _CH, _CH)]],
                    bufs[pb], gsems[pb])
            gh[b].wait()
            oh[b] = pltpu.async_copy(
                bufs[b], out_hbm.at[row, pl.ds(col0 + c * _CH, _CH)], osems[b])
        for c in range(max(0, _NCH - _NBUF), _NCH):
            oh[c % _NBUF].wait()

    return body(tokens, w_e)


def kernel(tokens, W_E):
    return _embed_lookup(tokens, W_E)
